# Initial kernel scaffold; baseline (speedup 1.0000x reference)
import jax
import jax.numpy as jnp
from jax.experimental import pallas as pl


def kernel(a, b, e, W1, b1, W2, b2):
    raise NotImplementedError("write your pallas kernel here")


# speedup vs baseline: 38.7703x; 38.7703x over previous
import functools

import jax
import jax.numpy as jnp
from jax import lax
from jax.experimental import pallas as pl
from jax.experimental.pallas import tpu as pltpu
from jax.experimental.pallas import tpu_sc as plsc

B, N, DEG, F = 8, 2048, 16, 128
R = B * N
NC, NS, L = 2, 16, 16
NW = NC * NS
RPW = R // NW
CH = 8
NCHUNK = RPW // CH
FC = F // L


def _reduce_chunk(gbuf, sbuf, obuf, is_max):
    def crow(c, _):
        gb = c * DEG
        for fc in range(FC):
            sl = pl.ds(fc * L, L)
            acc = sbuf[c, sl]
            for d in range(DEG):
                v = gbuf[gb + d, sl]
                acc = jnp.maximum(acc, v) if is_max else acc + v
            obuf[c, sl] = acc
        return 0
    lax.fori_loop(0, CH, crow, 0)


def _issue_loads(h_hbm, eg, ebuf, j, gbuf, sbuf, lsem, row0):
    pltpu.async_copy(h_hbm.at[ebuf.at[j]], gbuf, lsem)
    pltpu.async_copy(h_hbm.at[pl.ds(row0 + j * CH, CH)], sbuf, lsem)


def _wait_loads(h_hbm, ebuf, j, gbuf, sbuf, lsem, row0):
    pltpu.make_async_copy(h_hbm.at[ebuf.at[j]], gbuf, lsem).wait()
    pltpu.make_async_copy(h_hbm.at[pl.ds(row0 + j * CH, CH)], sbuf, lsem).wait()


def _gather_phase_body(h_hbm, eg_hbm, out_hbm,
                       ebuf, gbufA, gbufB, sbufA, sbufB, obufA, obufB,
                       lsemA, lsemB, osem, *, is_max):
    w = lax.axis_index("s") * NC + lax.axis_index("c")
    row0 = w * RPW
    pltpu.sync_copy(eg_hbm.at[pl.ds(w * NCHUNK, NCHUNK)], ebuf)
    _issue_loads(h_hbm, eg_hbm, ebuf, 0, gbufA, sbufA, lsemA, row0)
    _issue_loads(h_hbm, eg_hbm, ebuf, 1, gbufB, sbufB, lsemB, row0)

    def step(jj, _):
        j0 = jj * 2
        j1 = j0 + 1
        _wait_loads(h_hbm, ebuf, j0, gbufA, sbufA, lsemA, row0)
        _reduce_chunk(gbufA, sbufA, obufA, is_max)
        _issue_loads(h_hbm, eg_hbm, ebuf, j0 + 2, gbufA, sbufA, lsemA, row0)
        pltpu.async_copy(obufA, out_hbm.at[pl.ds(row0 + j0 * CH, CH)], osem).wait()
        _wait_loads(h_hbm, ebuf, j1, gbufB, sbufB, lsemB, row0)
        _reduce_chunk(gbufB, sbufB, obufB, is_max)
        _issue_loads(h_hbm, eg_hbm, ebuf, j1 + 2, gbufB, sbufB, lsemB, row0)
        pltpu.async_copy(obufB, out_hbm.at[pl.ds(row0 + j1 * CH, CH)], osem).wait()
        return 0

    lax.fori_loop(0, NCHUNK // 2 - 1, step, 0)
    j0 = NCHUNK - 2
    _wait_loads(h_hbm, ebuf, j0, gbufA, sbufA, lsemA, row0)
    _reduce_chunk(gbufA, sbufA, obufA, is_max)
    pltpu.async_copy(obufA, out_hbm.at[pl.ds(row0 + j0 * CH, CH)], osem).wait()
    j1 = NCHUNK - 1
    _wait_loads(h_hbm, ebuf, j1, gbufB, sbufB, lsemB, row0)
    _reduce_chunk(gbufB, sbufB, obufB, is_max)
    pltpu.async_copy(obufB, out_hbm.at[pl.ds(row0 + j1 * CH, CH)], osem).wait()


def _pool_sum_body(h_hbm, eg_hbm, out_hbm,
                   ebuf, gbufA, gbufB, sbufA, sbufB, accv,
                   lsemA, lsemB):
    w = lax.axis_index("s") * NC + lax.axis_index("c")
    row0 = w * RPW
    pltpu.sync_copy(eg_hbm.at[pl.ds(w * NCHUNK, NCHUNK)], ebuf)
    zero = jnp.zeros((L,), jnp.float32)
    for fc in range(FC):
        accv[pl.ds(fc * L, L)] = zero
    _issue_loads(h_hbm, eg_hbm, ebuf, 0, gbufA, sbufA, lsemA, row0)
    _issue_loads(h_hbm, eg_hbm, ebuf, 1, gbufB, sbufB, lsemB, row0)

    def pool_acc(gbuf, sbuf):
        def crow(c, _):
            gb = c * DEG
            for fc in range(FC):
                sl = pl.ds(fc * L, L)
                acc = sbuf[c, sl]
                for d in range(DEG):
                    acc = jnp.maximum(acc, gbuf[gb + d, sl])
                accv[sl] = accv[sl] + acc
            return 0
        lax.fori_loop(0, CH, crow, 0)

    def step(jj, _):
        j0 = jj * 2
        j1 = j0 + 1
        _wait_loads(h_hbm, ebuf, j0, gbufA, sbufA, lsemA, row0)
        pool_acc(gbufA, sbufA)
        _issue_loads(h_hbm, eg_hbm, ebuf, j0 + 2, gbufA, sbufA, lsemA, row0)
        _wait_loads(h_hbm, ebuf, j1, gbufB, sbufB, lsemB, row0)
        pool_acc(gbufB, sbufB)
        _issue_loads(h_hbm, eg_hbm, ebuf, j1 + 2, gbufB, sbufB, lsemB, row0)
        return 0

    lax.fori_loop(0, NCHUNK // 2 - 1, step, 0)
    _wait_loads(h_hbm, ebuf, NCHUNK - 2, gbufA, sbufA, lsemA, row0)
    pool_acc(gbufA, sbufA)
    _wait_loads(h_hbm, ebuf, NCHUNK - 1, gbufB, sbufB, lsemB, row0)
    pool_acc(gbufB, sbufB)
    pltpu.sync_copy(accv, out_hbm.at[w])


_GATHER_SCRATCH = [
    pltpu.VMEM((NCHUNK, 2 * DEG * CH // 2), jnp.int32),
    pltpu.VMEM((CH * DEG, F), jnp.float32),
    pltpu.VMEM((CH * DEG, F), jnp.float32),
    pltpu.VMEM((CH, F), jnp.float32),
    pltpu.VMEM((CH, F), jnp.float32),
    pltpu.VMEM((CH, F), jnp.float32),
    pltpu.VMEM((CH, F), jnp.float32),
    pltpu.SemaphoreType.DMA,
    pltpu.SemaphoreType.DMA,
    pltpu.SemaphoreType.DMA,
]

@functools.cache
def _sc_kernels():
    mesh = plsc.VectorSubcoreMesh(
        core_axis_name="c", subcore_axis_name="s",
        num_cores=NC, num_subcores=NS)
    gather_sum = functools.partial(
        pl.kernel,
        out_type=jax.ShapeDtypeStruct((R, F), jnp.float32),
        mesh=mesh,
        scratch_types=_GATHER_SCRATCH,
    )(functools.partial(_gather_phase_body, is_max=False))
    gather_max = functools.partial(
        pl.kernel,
        out_type=jax.ShapeDtypeStruct((R, F), jnp.float32),
        mesh=mesh,
        scratch_types=_GATHER_SCRATCH,
    )(functools.partial(_gather_phase_body, is_max=True))
    pool_sum = functools.partial(
        pl.kernel,
        out_type=jax.ShapeDtypeStruct((NW, F), jnp.float32),
        mesh=mesh,
        scratch_types=[
            pltpu.VMEM((NCHUNK, DEG * CH), jnp.int32),
            pltpu.VMEM((CH * DEG, F), jnp.float32),
            pltpu.VMEM((CH * DEG, F), jnp.float32),
            pltpu.VMEM((CH, F), jnp.float32),
            pltpu.VMEM((CH, F), jnp.float32),
            pltpu.VMEM((F,), jnp.float32),
            pltpu.SemaphoreType.DMA,
            pltpu.SemaphoreType.DMA,
        ],
    )(_pool_sum_body)
    return gather_sum, gather_max, pool_sum


def _mm_relu_body(x_ref, w_ref, b_ref, o_ref):
    o_ref[...] = jnp.maximum(
        jnp.dot(x_ref[...], w_ref[...], preferred_element_type=jnp.float32)
        + b_ref[...], 0.0)


_MM_ROWS = 1024

_mm_relu = pl.pallas_call(
    _mm_relu_body,
    grid=(R // _MM_ROWS,),
    in_specs=[
        pl.BlockSpec((_MM_ROWS, F), lambda i: (i, 0)),
        pl.BlockSpec((F, F), lambda i: (0, 0)),
        pl.BlockSpec((1, F), lambda i: (0, 0)),
    ],
    out_specs=pl.BlockSpec((_MM_ROWS, F), lambda i: (i, 0)),
    out_shape=jax.ShapeDtypeStruct((R, F), jnp.float32),
)


def kernel(a, b, e, W1, b1, W2, b2):
    del b
    a2 = a.reshape(R, F)
    eg = (e.astype(jnp.int32)
          + (jnp.arange(B, dtype=jnp.int32) * N)[:, None, None])
    eg2d = eg.reshape(R * DEG // 128, 128)
    gather_sum, gather_max, pool_sum = _sc_kernels()
    s1 = gather_sum(a2, eg2d)
    h1 = _mm_relu(s1, W1, b1.reshape(1, F))
    p1 = gather_max(h1, eg2d)
    s2 = gather_sum(p1, eg2d)
    h2 = _mm_relu(s2, W2, b2.reshape(1, F))
    part = pool_sum(h2, eg2d)
    return part.reshape(B, NW // B, F).sum(axis=1)


# speedup vs baseline: 45.1575x; 1.1647x over previous
import functools

import jax
import jax.numpy as jnp
from jax import lax
from jax.experimental import pallas as pl
from jax.experimental.pallas import tpu as pltpu
from jax.experimental.pallas import tpu_sc as plsc

B, N, DEG, F = 8, 2048, 16, 128
R = B * N
NC, NS, L = 2, 16, 16
NW = NC * NS
RPW = R // NW
CH = 8
NCHUNK = RPW // CH
FC = F // L


def _tree17(vals, op):
    while len(vals) > 1:
        nxt = [op(vals[i], vals[i + 1]) for i in range(0, len(vals) - 1, 2)]
        if len(vals) % 2:
            nxt.append(vals[-1])
        vals = nxt
    return vals[0]


def _reduce_chunk(gbuf, sbuf, obuf, is_max):
    op = jnp.maximum if is_max else jnp.add
    def crow(c, _):
        gb = c * DEG
        for fc in range(FC):
            sl = pl.ds(fc * L, L)
            vals = [sbuf[c, sl]] + [gbuf[gb + d, sl] for d in range(DEG)]
            obuf[c, sl] = _tree17(vals, op)
        return 0
    lax.fori_loop(0, CH, crow, 0)


def _issue_loads(h_hbm, eg, ebuf, j, gbuf, sbuf, lsem, row0):
    pltpu.async_copy(h_hbm.at[ebuf.at[j]], gbuf, lsem)
    pltpu.async_copy(h_hbm.at[pl.ds(row0 + j * CH, CH)], sbuf, lsem)


def _wait_loads(h_hbm, ebuf, j, gbuf, sbuf, lsem, row0):
    pltpu.make_async_copy(h_hbm.at[ebuf.at[j]], gbuf, lsem).wait()
    pltpu.make_async_copy(h_hbm.at[pl.ds(row0 + j * CH, CH)], sbuf, lsem).wait()


def _gather_phase_body(h_hbm, eg_hbm, out_hbm,
                       ebuf, gbufA, gbufB, sbufA, sbufB, obufA, obufB,
                       lsemA, lsemB, osem, *, is_max):
    w = lax.axis_index("s") * NC + lax.axis_index("c")
    row0 = w * RPW
    pltpu.sync_copy(eg_hbm.at[pl.ds(w * NCHUNK, NCHUNK)], ebuf)
    _issue_loads(h_hbm, eg_hbm, ebuf, 0, gbufA, sbufA, lsemA, row0)
    _issue_loads(h_hbm, eg_hbm, ebuf, 1, gbufB, sbufB, lsemB, row0)

    def step(jj, _):
        j0 = jj * 2
        j1 = j0 + 1
        _wait_loads(h_hbm, ebuf, j0, gbufA, sbufA, lsemA, row0)
        _reduce_chunk(gbufA, sbufA, obufA, is_max)
        _issue_loads(h_hbm, eg_hbm, ebuf, j0 + 2, gbufA, sbufA, lsemA, row0)
        pltpu.async_copy(obufA, out_hbm.at[pl.ds(row0 + j0 * CH, CH)], osem).wait()
        _wait_loads(h_hbm, ebuf, j1, gbufB, sbufB, lsemB, row0)
        _reduce_chunk(gbufB, sbufB, obufB, is_max)
        _issue_loads(h_hbm, eg_hbm, ebuf, j1 + 2, gbufB, sbufB, lsemB, row0)
        pltpu.async_copy(obufB, out_hbm.at[pl.ds(row0 + j1 * CH, CH)], osem).wait()
        return 0

    lax.fori_loop(0, NCHUNK // 2 - 1, step, 0)
    j0 = NCHUNK - 2
    _wait_loads(h_hbm, ebuf, j0, gbufA, sbufA, lsemA, row0)
    _reduce_chunk(gbufA, sbufA, obufA, is_max)
    pltpu.async_copy(obufA, out_hbm.at[pl.ds(row0 + j0 * CH, CH)], osem).wait()
    j1 = NCHUNK - 1
    _wait_loads(h_hbm, ebuf, j1, gbufB, sbufB, lsemB, row0)
    _reduce_chunk(gbufB, sbufB, obufB, is_max)
    pltpu.async_copy(obufB, out_hbm.at[pl.ds(row0 + j1 * CH, CH)], osem).wait()


def _pool_sum_body(h_hbm, eg_hbm, out_hbm,
                   ebuf, gbufA, gbufB, sbufA, sbufB, accv,
                   lsemA, lsemB):
    w = lax.axis_index("s") * NC + lax.axis_index("c")
    row0 = w * RPW
    pltpu.sync_copy(eg_hbm.at[pl.ds(w * NCHUNK, NCHUNK)], ebuf)
    zero = jnp.zeros((L,), jnp.float32)
    for fc in range(FC):
        accv[pl.ds(fc * L, L)] = zero
    _issue_loads(h_hbm, eg_hbm, ebuf, 0, gbufA, sbufA, lsemA, row0)
    _issue_loads(h_hbm, eg_hbm, ebuf, 1, gbufB, sbufB, lsemB, row0)

    def pool_acc(gbuf, sbuf):
        def crow(c, _):
            gb = c * DEG
            for fc in range(FC):
                sl = pl.ds(fc * L, L)
                vals = [sbuf[c, sl]] + [gbuf[gb + d, sl] for d in range(DEG)]
                accv[sl] = accv[sl] + _tree17(vals, jnp.maximum)
            return 0
        lax.fori_loop(0, CH, crow, 0)

    def step(jj, _):
        j0 = jj * 2
        j1 = j0 + 1
        _wait_loads(h_hbm, ebuf, j0, gbufA, sbufA, lsemA, row0)
        pool_acc(gbufA, sbufA)
        _issue_loads(h_hbm, eg_hbm, ebuf, j0 + 2, gbufA, sbufA, lsemA, row0)
        _wait_loads(h_hbm, ebuf, j1, gbufB, sbufB, lsemB, row0)
        pool_acc(gbufB, sbufB)
        _issue_loads(h_hbm, eg_hbm, ebuf, j1 + 2, gbufB, sbufB, lsemB, row0)
        return 0

    lax.fori_loop(0, NCHUNK // 2 - 1, step, 0)
    _wait_loads(h_hbm, ebuf, NCHUNK - 2, gbufA, sbufA, lsemA, row0)
    pool_acc(gbufA, sbufA)
    _wait_loads(h_hbm, ebuf, NCHUNK - 1, gbufB, sbufB, lsemB, row0)
    pool_acc(gbufB, sbufB)
    pltpu.sync_copy(accv, out_hbm.at[w])


_GATHER_SCRATCH = [
    pltpu.VMEM((NCHUNK, 2 * DEG * CH // 2), jnp.int32),
    pltpu.VMEM((CH * DEG, F), jnp.float32),
    pltpu.VMEM((CH * DEG, F), jnp.float32),
    pltpu.VMEM((CH, F), jnp.float32),
    pltpu.VMEM((CH, F), jnp.float32),
    pltpu.VMEM((CH, F), jnp.float32),
    pltpu.VMEM((CH, F), jnp.float32),
    pltpu.SemaphoreType.DMA,
    pltpu.SemaphoreType.DMA,
    pltpu.SemaphoreType.DMA,
]

@functools.cache
def _sc_kernels():
    mesh = plsc.VectorSubcoreMesh(
        core_axis_name="c", subcore_axis_name="s",
        num_cores=NC, num_subcores=NS)
    gather_sum = functools.partial(
        pl.kernel,
        out_type=jax.ShapeDtypeStruct((R, F), jnp.float32),
        mesh=mesh,
        scratch_types=_GATHER_SCRATCH,
    )(functools.partial(_gather_phase_body, is_max=False))
    gather_max = functools.partial(
        pl.kernel,
        out_type=jax.ShapeDtypeStruct((R, F), jnp.float32),
        mesh=mesh,
        scratch_types=_GATHER_SCRATCH,
    )(functools.partial(_gather_phase_body, is_max=True))
    pool_sum = functools.partial(
        pl.kernel,
        out_type=jax.ShapeDtypeStruct((NW, F), jnp.float32),
        mesh=mesh,
        scratch_types=[
            pltpu.VMEM((NCHUNK, DEG * CH), jnp.int32),
            pltpu.VMEM((CH * DEG, F), jnp.float32),
            pltpu.VMEM((CH * DEG, F), jnp.float32),
            pltpu.VMEM((CH, F), jnp.float32),
            pltpu.VMEM((CH, F), jnp.float32),
            pltpu.VMEM((F,), jnp.float32),
            pltpu.SemaphoreType.DMA,
            pltpu.SemaphoreType.DMA,
        ],
    )(_pool_sum_body)
    return gather_sum, gather_max, pool_sum


def _mm_relu_body(x_ref, w_ref, b_ref, o_ref):
    o_ref[...] = jnp.maximum(
        jnp.dot(x_ref[...], w_ref[...], preferred_element_type=jnp.float32)
        + b_ref[...], 0.0)


_MM_ROWS = 1024

_mm_relu = pl.pallas_call(
    _mm_relu_body,
    grid=(R // _MM_ROWS,),
    in_specs=[
        pl.BlockSpec((_MM_ROWS, F), lambda i: (i, 0)),
        pl.BlockSpec((F, F), lambda i: (0, 0)),
        pl.BlockSpec((1, F), lambda i: (0, 0)),
    ],
    out_specs=pl.BlockSpec((_MM_ROWS, F), lambda i: (i, 0)),
    out_shape=jax.ShapeDtypeStruct((R, F), jnp.float32),
)


def kernel(a, b, e, W1, b1, W2, b2):
    del b
    a2 = a.reshape(R, F)
    eg = (e.astype(jnp.int32)
          + (jnp.arange(B, dtype=jnp.int32) * N)[:, None, None])
    eg2d = eg.reshape(R * DEG // 128, 128)
    gather_sum, gather_max, pool_sum = _sc_kernels()
    s1 = gather_sum(a2, eg2d)
    h1 = _mm_relu(s1, W1, b1.reshape(1, F))
    p1 = gather_max(h1, eg2d)
    s2 = gather_sum(p1, eg2d)
    h2 = _mm_relu(s2, W2, b2.reshape(1, F))
    part = pool_sum(h2, eg2d)
    return part.reshape(B, NW // B, F).sum(axis=1)
